# baseline (speedup 1.0000x reference)
import jax
import jax.numpy as jnp
from jax.experimental import pallas as pl
from jax.experimental.pallas import tpu as pltpu

N_CODES = 1024
DIM = 64
BETA_C = 0.25
TOKENS = 32768
BLK = 1024
N_BLOCKS = TOKENS // BLK


def _vq_body(z_ref, w_ref, zq_ref, oh_ref, idx_ref, loss_ref,
             perp_ref, counts_ref, loss_acc_ref, wsq_ref):
    i = pl.program_id(0)

    @pl.when(i == 0)
    def _init():
        counts_ref[...] = jnp.zeros_like(counts_ref)
        loss_acc_ref[...] = jnp.zeros_like(loss_acc_ref)
        w0 = w_ref[...]
        wsq_ref[...] = jnp.sum(w0 * w0, axis=1, keepdims=True)

    z = z_ref[...]
    w = w_ref[...]

    ones_dim = jnp.ones((1, DIM), jnp.float32)
    zsq_row = jax.lax.dot_general(
        ones_dim, z * z, (((1,), (1,)), ((), ())),
        preferred_element_type=jnp.float32)
    wsq_col = wsq_ref[...]
    mmt = jax.lax.dot_general(
        w, z, (((1,), (1,)), ((), ())),
        preferred_element_type=jnp.float32)
    dt = (zsq_row + wsq_col) - 2.0 * mmt

    dmin = jnp.min(dt, axis=0, keepdims=True)
    code = jax.lax.broadcasted_iota(jnp.int32, (N_CODES, BLK), 0)
    masked = jnp.where(dt == dmin, code, N_CODES)
    idx = jnp.min(masked, axis=0)

    oh_t = (masked == idx[None, :]).astype(jnp.float32)
    zq = jax.lax.dot_general(
        oh_t, w, (((0,), (0,)), ((), ())),
        preferred_element_type=jnp.float32)

    diff = zq - z
    ones_row = jnp.ones((1, BLK), jnp.float32)
    counts_ref[...] += jax.lax.dot_general(
        ones_row, oh_t, (((1,), (1,)), ((), ())),
        preferred_element_type=jnp.float32)
    loss_acc_ref[...] += jax.lax.dot_general(
        ones_row, diff * diff, (((1,), (0,)), ((), ())),
        preferred_element_type=jnp.float32)

    oh_ref[...] = jnp.transpose(oh_t)
    zq_ref[...] = z + (zq - z)
    idx_ref[...] = idx.reshape(1, 1, BLK)

    @pl.when(i == N_BLOCKS - 1)
    def _finalize():
        m = jnp.sum(loss_acc_ref[...]) * (1.0 / (TOKENS * DIM))
        loss_ref[...] = jnp.reshape(m + BETA_C * m, (1, 1))
        e_mean = counts_ref[...] * (1.0 / TOKENS)
        ent = -jnp.sum(e_mean * jnp.log(e_mean + 1e-10))
        perp_ref[...] = jnp.reshape(jnp.exp(ent), (1, 1))


@jax.jit
def _vq(z, W):
    out_shape = (
        jax.ShapeDtypeStruct((TOKENS, DIM), jnp.float32),
        jax.ShapeDtypeStruct((TOKENS, N_CODES), jnp.float32),
        jax.ShapeDtypeStruct((N_BLOCKS, 1, BLK), jnp.int32),
        jax.ShapeDtypeStruct((1, 1), jnp.float32),
        jax.ShapeDtypeStruct((1, 1), jnp.float32),
    )
    grid = (N_BLOCKS,)
    zq, oh, idx, loss, perp = pl.pallas_call(
        _vq_body,
        grid=grid,
        in_specs=[
            pl.BlockSpec((BLK, DIM), lambda i: (i, 0)),
            pl.BlockSpec((N_CODES, DIM), lambda i: (0, 0)),
        ],
        out_specs=[
            pl.BlockSpec((BLK, DIM), lambda i: (i, 0)),
            pl.BlockSpec((BLK, N_CODES), lambda i: (i, 0)),
            pl.BlockSpec((1, 1, BLK), lambda i: (i, 0, 0)),
            pl.BlockSpec((1, 1), lambda i: (0, 0)),
            pl.BlockSpec((1, 1), lambda i: (0, 0)),
        ],
        out_shape=out_shape,
        scratch_shapes=[
            pltpu.VMEM((1, N_CODES), jnp.float32),
            pltpu.VMEM((1, DIM), jnp.float32),
            pltpu.VMEM((N_CODES, 1), jnp.float32),
        ],
    )(z, W)
    return zq, oh, idx, loss, perp


def kernel(z, W):
    zq, oh, idx, loss, perp = _vq(z, W)
    min_encoding_indices = idx.reshape(TOKENS, 1)
    return (loss[0, 0], zq, perp[0, 0], oh, min_encoding_indices)


# speedup vs baseline: 1.0590x; 1.0590x over previous
import jax
import jax.numpy as jnp
from jax.experimental import pallas as pl
from jax.experimental.pallas import tpu as pltpu

N_CODES = 1024
DIM = 64
BETA_C = 0.25
TOKENS = 32768
BLK = 1024
N_BLOCKS = TOKENS // BLK


def _vq_body(z_ref, w_ref, zq_ref, oh_ref, idx_ref, loss_ref,
             perp_ref, counts_ref, loss_acc_ref, wsq_ref):
    i = pl.program_id(0)

    @pl.when(i == 0)
    def _init():
        counts_ref[...] = jnp.zeros_like(counts_ref)
        loss_acc_ref[...] = jnp.zeros_like(loss_acc_ref)
        w0 = w_ref[...]
        wsq_ref[...] = jnp.sum(w0 * w0, axis=1)[None, :]

    z = z_ref[...]
    w = w_ref[...]

    zsq = jnp.sum(z * z, axis=1, keepdims=True)
    wsq = wsq_ref[...]
    mm = jax.lax.dot_general(
        z, w, (((1,), (1,)), ((), ())),
        preferred_element_type=jnp.float32)
    d = (zsq + wsq) - 2.0 * mm

    LW = 128
    NCOL = N_CODES // LW
    cols = [d[:, c * LW:(c + 1) * LW] for c in range(NCOL)]
    dmin128 = cols[0]
    for c in range(1, NCOL):
        dmin128 = jnp.minimum(dmin128, cols[c])
    dmin = jnp.min(dmin128, axis=1, keepdims=True)
    dminb = jnp.broadcast_to(dmin, (BLK, LW))
    lane128 = jax.lax.broadcasted_iota(jnp.int32, (BLK, LW), 1)
    cand = None
    for c in range(NCOL):
        cc = jnp.where(cols[c] == dminb, lane128 + c * LW, N_CODES)
        cand = cc if cand is None else jnp.minimum(cand, cc)
    idx = jnp.min(cand, axis=1)

    lane = jax.lax.broadcasted_iota(jnp.int32, (BLK, N_CODES), 1)

    one_hot = (lane == idx[:, None]).astype(jnp.float32)
    zq = jax.lax.dot_general(
        one_hot, w, (((1,), (0,)), ((), ())),
        preferred_element_type=jnp.float32)

    diff = zq - z
    ones_row = jnp.ones((1, BLK), jnp.float32)
    counts_ref[...] += jax.lax.dot_general(
        ones_row, one_hot, (((1,), (0,)), ((), ())),
        preferred_element_type=jnp.float32)
    loss_acc_ref[...] += jax.lax.dot_general(
        ones_row, diff * diff, (((1,), (0,)), ((), ())),
        preferred_element_type=jnp.float32)

    oh_ref[...] = one_hot
    zq_ref[...] = z + (zq - z)
    idx_ref[...] = idx.reshape(1, 1, BLK)

    @pl.when(i == N_BLOCKS - 1)
    def _finalize():
        m = jnp.sum(loss_acc_ref[...]) * (1.0 / (TOKENS * DIM))
        loss_ref[...] = jnp.reshape(m + BETA_C * m, (1, 1))
        e_mean = counts_ref[...] * (1.0 / TOKENS)
        ent = -jnp.sum(e_mean * jnp.log(e_mean + 1e-10))
        perp_ref[...] = jnp.reshape(jnp.exp(ent), (1, 1))


def _vq(z, W):
    out_shape = (
        jax.ShapeDtypeStruct((TOKENS, DIM), jnp.float32),
        jax.ShapeDtypeStruct((TOKENS, N_CODES), jnp.float32),
        jax.ShapeDtypeStruct((N_BLOCKS, 1, BLK), jnp.int32),
        jax.ShapeDtypeStruct((1, 1), jnp.float32),
        jax.ShapeDtypeStruct((1, 1), jnp.float32),
    )
    grid = (N_BLOCKS,)
    zq, oh, idx, loss, perp = pl.pallas_call(
        _vq_body,
        grid=grid,
        in_specs=[
            pl.BlockSpec((BLK, DIM), lambda i: (i, 0)),
            pl.BlockSpec((N_CODES, DIM), lambda i: (0, 0)),
        ],
        out_specs=[
            pl.BlockSpec((BLK, DIM), lambda i: (i, 0)),
            pl.BlockSpec((BLK, N_CODES), lambda i: (i, 0)),
            pl.BlockSpec((1, 1, BLK), lambda i: (i, 0, 0)),
            pl.BlockSpec((1, 1), lambda i: (0, 0)),
            pl.BlockSpec((1, 1), lambda i: (0, 0)),
        ],
        out_shape=out_shape,
        scratch_shapes=[
            pltpu.VMEM((1, N_CODES), jnp.float32),
            pltpu.VMEM((1, DIM), jnp.float32),
            pltpu.VMEM((1, N_CODES), jnp.float32),
        ],
    )(z, W)
    return zq, oh, idx, loss, perp


def kernel(z, W):
    zq, oh, idx, loss, perp = _vq(z, W)
    min_encoding_indices = idx.reshape(TOKENS, 1)
    return (loss[0, 0], zq, perp[0, 0], oh, min_encoding_indices)


# speedup vs baseline: 1.2114x; 1.1439x over previous
import jax
import jax.numpy as jnp
from jax.experimental import pallas as pl
from jax.experimental.pallas import tpu as pltpu

N_CODES = 1024
DIM = 64
BETA_C = 0.25
TOKENS = 32768
BLK = 1024
N_BLOCKS = TOKENS // BLK


def _vq_body(zt_ref, wt_ref, zqt_ref, oh_ref, idx_ref, loss_ref,
             perp_ref, counts_ref, loss_acc_ref, wsq_ref):
    i = pl.program_id(0)

    @pl.when(i == 0)
    def _init():
        counts_ref[...] = jnp.zeros_like(counts_ref)
        loss_acc_ref[...] = jnp.zeros_like(loss_acc_ref)
        w0 = wt_ref[...]
        wsq_ref[...] = jnp.transpose(
            jnp.sum(w0 * w0, axis=0, keepdims=True))

    zt = zt_ref[...]
    wt = wt_ref[...]

    ones_dim = jnp.ones((1, DIM), jnp.float32)
    zsq_row = jax.lax.dot_general(
        ones_dim, zt * zt, (((1,), (0,)), ((), ())),
        preferred_element_type=jnp.float32)
    wsq_col = wsq_ref[...]
    mmt = jax.lax.dot_general(
        wt, zt, (((0,), (0,)), ((), ())),
        preferred_element_type=jnp.float32)
    dt = (zsq_row + wsq_col) - 2.0 * mmt

    dmin = jnp.min(dt, axis=0, keepdims=True)
    code = jax.lax.broadcasted_iota(jnp.int32, (N_CODES, BLK), 0)
    masked = jnp.where(dt == dmin, code, N_CODES)
    idx = jnp.min(masked, axis=0)

    oh_t = (masked == idx[None, :]).astype(jnp.float32)
    zqt = jax.lax.dot_general(
        wt, oh_t, (((1,), (0,)), ((), ())),
        preferred_element_type=jnp.float32)

    diff = zqt - zt
    counts_ref[...] += jax.lax.dot_general(
        oh_t, jnp.ones((BLK, 1), jnp.float32), (((1,), (0,)), ((), ())),
        preferred_element_type=jnp.float32)
    loss_acc_ref[...] += jax.lax.dot_general(
        ones_dim, diff * diff, (((1,), (0,)), ((), ())),
        preferred_element_type=jnp.float32)

    oh_ref[...] = jnp.transpose(oh_t)
    zqt_ref[...] = zt + (zqt - zt)
    idx_ref[...] = idx.reshape(1, 1, BLK)

    @pl.when(i == N_BLOCKS - 1)
    def _finalize():
        m = jnp.sum(loss_acc_ref[...]) * (1.0 / (TOKENS * DIM))
        loss_ref[...] = jnp.reshape(m + BETA_C * m, (1, 1))
        e_mean = counts_ref[...] * (1.0 / TOKENS)
        ent = -jnp.sum(e_mean * jnp.log(e_mean + 1e-10))
        perp_ref[...] = jnp.reshape(jnp.exp(ent), (1, 1))


def _vq(zt, wt):
    out_shape = (
        jax.ShapeDtypeStruct((DIM, TOKENS), jnp.float32),
        jax.ShapeDtypeStruct((TOKENS, N_CODES), jnp.float32),
        jax.ShapeDtypeStruct((N_BLOCKS, 1, BLK), jnp.int32),
        jax.ShapeDtypeStruct((1, 1), jnp.float32),
        jax.ShapeDtypeStruct((1, 1), jnp.float32),
    )
    grid = (N_BLOCKS,)
    zqt, oh, idx, loss, perp = pl.pallas_call(
        _vq_body,
        grid=grid,
        in_specs=[
            pl.BlockSpec((DIM, BLK), lambda i: (0, i)),
            pl.BlockSpec((DIM, N_CODES), lambda i: (0, 0)),
        ],
        out_specs=[
            pl.BlockSpec((DIM, BLK), lambda i: (0, i)),
            pl.BlockSpec((BLK, N_CODES), lambda i: (i, 0)),
            pl.BlockSpec((1, 1, BLK), lambda i: (i, 0, 0)),
            pl.BlockSpec((1, 1), lambda i: (0, 0)),
            pl.BlockSpec((1, 1), lambda i: (0, 0)),
        ],
        out_shape=out_shape,
        scratch_shapes=[
            pltpu.VMEM((N_CODES, 1), jnp.float32),
            pltpu.VMEM((1, BLK), jnp.float32),
            pltpu.VMEM((N_CODES, 1), jnp.float32),
        ],
    )(zt, wt)
    return zqt, oh, idx, loss, perp


def kernel(z, W):
    zqt, oh, idx, loss, perp = _vq(z.T, W.T)
    min_encoding_indices = idx.reshape(TOKENS, 1)
    return (loss[0, 0], zqt.T, perp[0, 0], oh, min_encoding_indices)


# speedup vs baseline: 1.3171x; 1.0872x over previous
import jax
import jax.numpy as jnp
from jax.experimental import pallas as pl
from jax.experimental.pallas import tpu as pltpu

N_CODES = 1024
DIM = 64
BETA_C = 0.25
TOKENS = 32768
BLK = 2048
N_BLOCKS = TOKENS // BLK


def _vq_body(zt_ref, wt_ref, zqt_ref, oh_ref, idx_ref, loss_ref,
             perp_ref, counts_ref, loss_acc_ref, wsq_ref):
    i = pl.program_id(0)

    @pl.when(i == 0)
    def _init():
        counts_ref[...] = jnp.zeros_like(counts_ref)
        loss_acc_ref[...] = jnp.zeros_like(loss_acc_ref)
        w0 = wt_ref[...]
        wsq_ref[...] = jnp.transpose(
            jnp.sum(w0 * w0, axis=0, keepdims=True))

    zt = zt_ref[...]
    wt = wt_ref[...]

    ones_dim = jnp.ones((1, DIM), jnp.float32)
    zsq_row = jax.lax.dot_general(
        ones_dim, zt * zt, (((1,), (0,)), ((), ())),
        preferred_element_type=jnp.float32)
    wsq_col = wsq_ref[...]
    mmt = jax.lax.dot_general(
        wt, zt, (((0,), (0,)), ((), ())),
        preferred_element_type=jnp.float32)
    dt = (zsq_row + wsq_col) - 2.0 * mmt

    dmin = jnp.min(dt, axis=0, keepdims=True)
    code = jax.lax.broadcasted_iota(jnp.int32, (N_CODES, BLK), 0)
    masked = jnp.where(dt == dmin, code, N_CODES)
    idx = jnp.min(masked, axis=0)

    oh_t = (masked == idx[None, :]).astype(jnp.float32)
    zqt = jax.lax.dot_general(
        wt, oh_t, (((1,), (0,)), ((), ())),
        preferred_element_type=jnp.float32)

    diff = zqt - zt
    counts_ref[...] += jax.lax.dot_general(
        oh_t, jnp.ones((BLK, 1), jnp.float32), (((1,), (0,)), ((), ())),
        preferred_element_type=jnp.float32)
    loss_acc_ref[...] += jax.lax.dot_general(
        ones_dim, diff * diff, (((1,), (0,)), ((), ())),
        preferred_element_type=jnp.float32)

    oh_ref[...] = jnp.transpose(oh_t)
    zqt_ref[...] = zt + (zqt - zt)
    idx_ref[...] = idx.reshape(1, 1, BLK)

    @pl.when(i == N_BLOCKS - 1)
    def _finalize():
        m = jnp.sum(loss_acc_ref[...]) * (1.0 / (TOKENS * DIM))
        loss_ref[...] = jnp.reshape(m + BETA_C * m, (1, 1))
        e_mean = counts_ref[...] * (1.0 / TOKENS)
        ent = -jnp.sum(e_mean * jnp.log(e_mean + 1e-10))
        perp_ref[...] = jnp.reshape(jnp.exp(ent), (1, 1))


def _vq(zt, wt):
    out_shape = (
        jax.ShapeDtypeStruct((DIM, TOKENS), jnp.float32),
        jax.ShapeDtypeStruct((TOKENS, N_CODES), jnp.float32),
        jax.ShapeDtypeStruct((N_BLOCKS, 1, BLK), jnp.int32),
        jax.ShapeDtypeStruct((1, 1), jnp.float32),
        jax.ShapeDtypeStruct((1, 1), jnp.float32),
    )
    grid = (N_BLOCKS,)
    zqt, oh, idx, loss, perp = pl.pallas_call(
        _vq_body,
        grid=grid,
        in_specs=[
            pl.BlockSpec((DIM, BLK), lambda i: (0, i)),
            pl.BlockSpec((DIM, N_CODES), lambda i: (0, 0)),
        ],
        out_specs=[
            pl.BlockSpec((DIM, BLK), lambda i: (0, i)),
            pl.BlockSpec((BLK, N_CODES), lambda i: (i, 0)),
            pl.BlockSpec((1, 1, BLK), lambda i: (i, 0, 0)),
            pl.BlockSpec((1, 1), lambda i: (0, 0)),
            pl.BlockSpec((1, 1), lambda i: (0, 0)),
        ],
        out_shape=out_shape,
        scratch_shapes=[
            pltpu.VMEM((N_CODES, 1), jnp.float32),
            pltpu.VMEM((1, BLK), jnp.float32),
            pltpu.VMEM((N_CODES, 1), jnp.float32),
        ],
    )(zt, wt)
    return zqt, oh, idx, loss, perp


def kernel(z, W):
    zqt, oh, idx, loss, perp = _vq(z.T, W.T)
    min_encoding_indices = idx.reshape(TOKENS, 1)
    return (loss[0, 0], zqt.T, perp[0, 0], oh, min_encoding_indices)
